# baseline (speedup 1.0000x reference)
import functools

import jax
import jax.numpy as jnp
from jax import lax
from jax.experimental import pallas as pl
from jax.experimental.pallas import tpu as pltpu
from jax.experimental.pallas import tpu_sc as plsc

N = 100000
E = 6400000

DT = 0.1
G_L = 10.0
DELTA_T = 2.0
V_THRESH = -50.0
V_REST = -65.0
CAP = 200.0
A_W = 2.0
TAU_W = 100.0
TAU_GE = 5.0
TAU_GI = 10.0
E_GE = 0.0
E_GI = -80.0
V_CUT = -30.0

NC = 2
NS = 16
NW = NC * NS
CHUNK = 2048
ROWS = CHUNK // 128
NCHUNKS = E // CHUNK
CPW = (NCHUNKS + NW - 1) // NW
CLS_STRIDE = 102400
CNT_LEN = 2 * CLS_STRIDE
ZSHARE = CNT_LEN // NS
ZCHUNK = 1600
INH_CODE = 2 * CLS_STRIDE + 1


def _code_body(spk_ref, exc_ref, code_ref):
    code_ref[...] = spk_ref[...] * (1 + (1 - exc_ref[...]) * (INH_CODE - 1))


def _edge_body(code_hbm, ei_hbm, out_hbm,
               code_v, src_v, dst_v, idx_v0, idx_v1, ones_v,
               zbuf, cnt_sh, in_sem0, in_sem1, scat_sem):
    cid = lax.axis_index("c")
    sid = lax.axis_index("s")
    wid = sid * NC + cid
    in_sems = (in_sem0, in_sem1)
    idx_bufs = (idx_v0, idx_v1)

    def ofill(i, _):
        ones_v[pl.ds(i * 16, 16)] = jnp.ones((16,), jnp.float32)
        return 0
    lax.fori_loop(0, CHUNK // 16, ofill, 0)

    def zfill(i, _):
        zbuf[pl.ds(i * 16, 16)] = jnp.zeros((16,), jnp.float32)
        return 0
    lax.fori_loop(0, ZCHUNK // 16, zfill, 0)
    for k in range(ZSHARE // ZCHUNK):
        pltpu.sync_copy(zbuf, cnt_sh.at[pl.ds(sid * ZSHARE + k * ZCHUNK, ZCHUNK)])

    code_desc = pltpu.async_copy(code_hbm, code_v, scat_sem)
    for b in (0, 1):
        off = (wid + b * NW) * CHUNK
        pltpu.async_copy(ei_hbm.at[0, pl.ds(off, CHUNK)], src_v.at[b], in_sems[b])
        pltpu.async_copy(ei_hbm.at[1, pl.ds(off, CHUNK)], dst_v.at[b], in_sems[b])
    code_desc.wait()
    plsc.subcore_barrier()

    def do_chunk(ci, b):
        g = wid + ci * NW
        idx_v = idx_bufs[b]

        @pl.when(g < NCHUNKS)
        def _process():
            off = g * CHUNK
            pltpu.make_async_copy(
                ei_hbm.at[0, pl.ds(off, CHUNK)], src_v.at[b], in_sems[b]).wait()
            pltpu.make_async_copy(
                ei_hbm.at[1, pl.ds(off, CHUNK)], dst_v.at[b], in_sems[b]).wait()
            @pl.when(ci >= 2)
            def _drain():
                pltpu.make_async_copy(
                    ones_v,
                    cnt_sh.at[plsc.Indices(idx_v, ignored_value=-1)],
                    scat_sem).wait()

            @plsc.parallel_loop(0, CHUNK // 16, 1, unroll=16)
            def _vloop(i):
                s = src_v[b, pl.ds(i * 16, 16)]
                d = dst_v[b, pl.ds(i * 16, 16)]
                t = plsc.load_gather(code_v, [s])
                slot = d + lax.shift_right_logical(t, 1)
                idx_v[pl.ds(i * 16, 16)] = jnp.where(
                    t == 0, jnp.int32(-1), slot)
            ng = g + 2 * NW

            @pl.when(ng < NCHUNKS)
            def _prefetch():
                noff = ng * CHUNK
                pltpu.async_copy(
                    ei_hbm.at[0, pl.ds(noff, CHUNK)], src_v.at[b], in_sems[b])
                pltpu.async_copy(
                    ei_hbm.at[1, pl.ds(noff, CHUNK)], dst_v.at[b], in_sems[b])
            pltpu.async_copy(
                ones_v,
                cnt_sh.at[plsc.Indices(idx_v, ignored_value=-1)],
                scat_sem, add=True)

    def outer_body(o, _):
        do_chunk(2 * o, 0)
        do_chunk(2 * o + 1, 1)
        return 0
    lax.fori_loop(0, CPW // 2, outer_body, 0)

    for ci in (CPW - 2, CPW - 1):
        b = ci % 2

        @pl.when(wid + ci * NW < NCHUNKS)
        def _final_drain():
            pltpu.make_async_copy(
                ones_v,
                cnt_sh.at[plsc.Indices(idx_bufs[b], ignored_value=-1)],
                scat_sem).wait()

    plsc.subcore_barrier()
    pltpu.sync_copy(cnt_sh.at[pl.ds(sid * ZSHARE, ZSHARE)],
                    out_hbm.at[cid, pl.ds(sid * ZSHARE, ZSHARE)])


_edge_kernel = functools.partial(
    pl.kernel,
    out_type=jax.ShapeDtypeStruct((NC, CNT_LEN), jnp.float32),
    mesh=plsc.VectorSubcoreMesh(core_axis_name="c", subcore_axis_name="s"),
    compiler_params=pltpu.CompilerParams(needs_layout_passes=False),
    scratch_types=[
        pltpu.VMEM((N,), jnp.int32),
        pltpu.VMEM((2, CHUNK), jnp.int32),
        pltpu.VMEM((2, CHUNK), jnp.int32),
        pltpu.VMEM((CHUNK,), jnp.int32),
        pltpu.VMEM((CHUNK,), jnp.int32),
        pltpu.VMEM((CHUNK,), jnp.float32),
        pltpu.VMEM((ZCHUNK,), jnp.float32),
        pltpu.VMEM_SHARED((CNT_LEN,), jnp.float32),
        pltpu.SemaphoreType.DMA,
        pltpu.SemaphoreType.DMA,
        pltpu.SemaphoreType.DMA,
    ],
)(_edge_body)


def _adex_body(v_ref, st_ref, w_ref, ge_ref, gi_ref, qge_ref, qgi_ref,
               ib_ref, vr_ref, b_ref, tr_ref, rc_ref,
               e0_ref, e1_ref, i0_ref, i1_ref,
               out_ref, spk_ref):
    v = v_ref[...]
    w = w_ref[...]
    ge = ge_ref[...] + qge_ref[...] * (e0_ref[...] + e1_ref[...])
    gi = gi_ref[...] + qgi_ref[...] * (i0_ref[...] + i1_ref[...])
    I = ib_ref[...] + st_ref[...] + ge * (E_GE - v) + gi * (E_GI - v)
    exp_term = G_L * DELTA_T * jnp.exp(
        jnp.minimum((v - V_THRESH) / DELTA_T, 20.0))
    dv = (-G_L * (v - V_REST) + exp_term - w + I) / CAP
    dw = (-w + A_W * (v - V_REST)) / TAU_W
    non_ref = rc_ref[...] <= 0.0
    v = jnp.where(non_ref, v + dv * DT, v)
    w = w + dw * DT
    ge = ge + (-ge / TAU_GE) * DT
    gi = gi + (-gi / TAU_GI) * DT
    new_spiked = v > V_CUT
    v = jnp.where(new_spiked, vr_ref[...], v)
    w = jnp.where(new_spiked, w + b_ref[...], w)
    refrac = jnp.where(new_spiked, tr_ref[...], rc_ref[...]) - DT
    out_ref[0, :] = v
    out_ref[1, :] = w
    out_ref[2, :] = ge
    out_ref[3, :] = gi
    out_ref[4, :] = refrac
    spk_ref[...] = new_spiked


def kernel(voltage, stimulus, adapt_current, ge, gi, Q_ge, Q_gi, I_bias,
           v_reset, b, t_refrac, refractory_counter, spiked, edge_index,
           is_excitatory):
    spk_i32 = spiked.astype(jnp.int32)
    exc_i32 = is_excitatory.astype(jnp.int32)
    ei = edge_index.astype(jnp.int32)

    code = pl.pallas_call(
        _code_body,
        out_shape=jax.ShapeDtypeStruct((N,), jnp.int32),
    )(spk_i32, exc_i32)

    pcounts = _edge_kernel(code, ei)

    e0 = pcounts[0, :N]
    e1 = pcounts[1, :N]
    i0 = pcounts[0, CLS_STRIDE:CLS_STRIDE + N]
    i1 = pcounts[1, CLS_STRIDE:CLS_STRIDE + N]

    out, new_spiked = pl.pallas_call(
        _adex_body,
        out_shape=(
            jax.ShapeDtypeStruct((5, N), jnp.float32),
            jax.ShapeDtypeStruct((N,), jnp.bool_),
        ),
    )(voltage, stimulus, adapt_current, ge, gi, Q_ge, Q_gi, I_bias,
      v_reset, b, t_refrac, refractory_counter, e0, e1, i0, i1)
    return out, new_spiked


# speedup vs baseline: 1.0016x; 1.0016x over previous
import functools

import jax
import jax.numpy as jnp
from jax import lax
from jax.experimental import pallas as pl
from jax.experimental.pallas import tpu as pltpu
from jax.experimental.pallas import tpu_sc as plsc

N = 100000
E = 6400000

DT = 0.1
G_L = 10.0
DELTA_T = 2.0
V_THRESH = -50.0
V_REST = -65.0
CAP = 200.0
A_W = 2.0
TAU_W = 100.0
TAU_GE = 5.0
TAU_GI = 10.0
E_GE = 0.0
E_GI = -80.0
V_CUT = -30.0

NC = 2
NS = 16
NW = NC * NS
CHUNK = 2048
ROWS = CHUNK // 128
NCHUNKS = E // CHUNK
CPW = (NCHUNKS + NW - 1) // NW
CLS_STRIDE = 102400
CNT_LEN = 2 * CLS_STRIDE
ZSHARE = CNT_LEN // NS
ZCHUNK = 1600
INH_CODE = 2 * CLS_STRIDE + 1


def _code_body(spk_ref, exc_ref, code_ref):
    code_ref[...] = spk_ref[...] * (1 + (1 - exc_ref[...]) * (INH_CODE - 1))


def _edge_body(code_hbm, ei_hbm, out_hbm,
               code_v, src_v, dst_v, idx_v0, idx_v1, ones_v,
               zbuf, cnt_sh, in_sem0, in_sem1, scat_sem):
    cid = lax.axis_index("c")
    sid = lax.axis_index("s")
    wid = sid * NC + cid
    in_sems = (in_sem0, in_sem1)
    idx_bufs = (idx_v0, idx_v1)

    def ofill(i, _):
        ones_v[pl.ds(i * 16, 16)] = jnp.ones((16,), jnp.float32)
        return 0
    lax.fori_loop(0, CHUNK // 16, ofill, 0)

    def zfill(i, _):
        zbuf[pl.ds(i * 16, 16)] = jnp.zeros((16,), jnp.float32)
        return 0
    lax.fori_loop(0, ZCHUNK // 16, zfill, 0)
    for k in range(ZSHARE // ZCHUNK):
        pltpu.sync_copy(zbuf, cnt_sh.at[pl.ds(sid * ZSHARE + k * ZCHUNK, ZCHUNK)])

    code_desc = pltpu.async_copy(code_hbm, code_v, scat_sem)
    for b in (0, 1):
        off = (wid + b * NW) * CHUNK
        pltpu.async_copy(ei_hbm.at[0, pl.ds(off, CHUNK)], src_v.at[b], in_sems[b])
        pltpu.async_copy(ei_hbm.at[1, pl.ds(off, CHUNK)], dst_v.at[b], in_sems[b])
    code_desc.wait()
    plsc.subcore_barrier()

    def do_chunk(ci, b):
        g = wid + ci * NW
        idx_v = idx_bufs[b]

        @pl.when(g < NCHUNKS)
        def _process():
            off = g * CHUNK
            pltpu.make_async_copy(
                ei_hbm.at[0, pl.ds(off, CHUNK)], src_v.at[b], in_sems[b]).wait()
            pltpu.make_async_copy(
                ei_hbm.at[1, pl.ds(off, CHUNK)], dst_v.at[b], in_sems[b]).wait()
            @pl.when(ci >= 2)
            def _drain():
                pltpu.make_async_copy(
                    ones_v,
                    cnt_sh.at[plsc.Indices(idx_v, ignored_value=-1)],
                    scat_sem).wait()

            @plsc.parallel_loop(0, CHUNK // 16, 1, unroll=8)
            def _vloop(i):
                s = src_v[b, pl.ds(i * 16, 16)]
                d = dst_v[b, pl.ds(i * 16, 16)]
                t = plsc.load_gather(code_v, [s])
                slot = d + lax.shift_right_logical(t, 1)
                idx_v[pl.ds(i * 16, 16)] = jnp.where(
                    t == 0, jnp.int32(-1), slot)
            ng = g + 2 * NW

            @pl.when(ng < NCHUNKS)
            def _prefetch():
                noff = ng * CHUNK
                pltpu.async_copy(
                    ei_hbm.at[0, pl.ds(noff, CHUNK)], src_v.at[b], in_sems[b])
                pltpu.async_copy(
                    ei_hbm.at[1, pl.ds(noff, CHUNK)], dst_v.at[b], in_sems[b])
            pltpu.async_copy(
                ones_v,
                cnt_sh.at[plsc.Indices(idx_v, ignored_value=-1)],
                scat_sem, add=True)

    def outer_body(o, _):
        do_chunk(2 * o, 0)
        do_chunk(2 * o + 1, 1)
        return 0
    lax.fori_loop(0, CPW // 2, outer_body, 0)

    for ci in (CPW - 2, CPW - 1):
        b = ci % 2

        @pl.when(wid + ci * NW < NCHUNKS)
        def _final_drain():
            pltpu.make_async_copy(
                ones_v,
                cnt_sh.at[plsc.Indices(idx_bufs[b], ignored_value=-1)],
                scat_sem).wait()

    plsc.subcore_barrier()
    pltpu.sync_copy(cnt_sh.at[pl.ds(sid * ZSHARE, ZSHARE)],
                    out_hbm.at[cid, pl.ds(sid * ZSHARE, ZSHARE)])


_edge_kernel = functools.partial(
    pl.kernel,
    out_type=jax.ShapeDtypeStruct((NC, CNT_LEN), jnp.float32),
    mesh=plsc.VectorSubcoreMesh(core_axis_name="c", subcore_axis_name="s"),
    compiler_params=pltpu.CompilerParams(needs_layout_passes=False),
    scratch_types=[
        pltpu.VMEM((N,), jnp.int32),
        pltpu.VMEM((2, CHUNK), jnp.int32),
        pltpu.VMEM((2, CHUNK), jnp.int32),
        pltpu.VMEM((CHUNK,), jnp.int32),
        pltpu.VMEM((CHUNK,), jnp.int32),
        pltpu.VMEM((CHUNK,), jnp.float32),
        pltpu.VMEM((ZCHUNK,), jnp.float32),
        pltpu.VMEM_SHARED((CNT_LEN,), jnp.float32),
        pltpu.SemaphoreType.DMA,
        pltpu.SemaphoreType.DMA,
        pltpu.SemaphoreType.DMA,
    ],
)(_edge_body)


def _adex_body(v_ref, st_ref, w_ref, ge_ref, gi_ref, qge_ref, qgi_ref,
               ib_ref, vr_ref, b_ref, tr_ref, rc_ref,
               e0_ref, e1_ref, i0_ref, i1_ref,
               out_ref, spk_ref):
    v = v_ref[...]
    w = w_ref[...]
    ge = ge_ref[...] + qge_ref[...] * (e0_ref[...] + e1_ref[...])
    gi = gi_ref[...] + qgi_ref[...] * (i0_ref[...] + i1_ref[...])
    I = ib_ref[...] + st_ref[...] + ge * (E_GE - v) + gi * (E_GI - v)
    exp_term = G_L * DELTA_T * jnp.exp(
        jnp.minimum((v - V_THRESH) / DELTA_T, 20.0))
    dv = (-G_L * (v - V_REST) + exp_term - w + I) / CAP
    dw = (-w + A_W * (v - V_REST)) / TAU_W
    non_ref = rc_ref[...] <= 0.0
    v = jnp.where(non_ref, v + dv * DT, v)
    w = w + dw * DT
    ge = ge + (-ge / TAU_GE) * DT
    gi = gi + (-gi / TAU_GI) * DT
    new_spiked = v > V_CUT
    v = jnp.where(new_spiked, vr_ref[...], v)
    w = jnp.where(new_spiked, w + b_ref[...], w)
    refrac = jnp.where(new_spiked, tr_ref[...], rc_ref[...]) - DT
    out_ref[0, :] = v
    out_ref[1, :] = w
    out_ref[2, :] = ge
    out_ref[3, :] = gi
    out_ref[4, :] = refrac
    spk_ref[...] = new_spiked


def kernel(voltage, stimulus, adapt_current, ge, gi, Q_ge, Q_gi, I_bias,
           v_reset, b, t_refrac, refractory_counter, spiked, edge_index,
           is_excitatory):
    spk_i32 = spiked.astype(jnp.int32)
    exc_i32 = is_excitatory.astype(jnp.int32)
    ei = edge_index.astype(jnp.int32)

    code = pl.pallas_call(
        _code_body,
        out_shape=jax.ShapeDtypeStruct((N,), jnp.int32),
    )(spk_i32, exc_i32)

    pcounts = _edge_kernel(code, ei)

    e0 = pcounts[0, :N]
    e1 = pcounts[1, :N]
    i0 = pcounts[0, CLS_STRIDE:CLS_STRIDE + N]
    i1 = pcounts[1, CLS_STRIDE:CLS_STRIDE + N]

    out, new_spiked = pl.pallas_call(
        _adex_body,
        out_shape=(
            jax.ShapeDtypeStruct((5, N), jnp.float32),
            jax.ShapeDtypeStruct((N,), jnp.bool_),
        ),
    )(voltage, stimulus, adapt_current, ge, gi, Q_ge, Q_gi, I_bias,
      v_reset, b, t_refrac, refractory_counter, e0, e1, i0, i1)
    return out, new_spiked


# speedup vs baseline: 1.0801x; 1.0783x over previous
import functools

import jax
import jax.numpy as jnp
from jax import lax
from jax.experimental import pallas as pl
from jax.experimental.pallas import tpu as pltpu
from jax.experimental.pallas import tpu_sc as plsc

N = 100000
E = 6400000

DT = 0.1
G_L = 10.0
DELTA_T = 2.0
V_THRESH = -50.0
V_REST = -65.0
CAP = 200.0
A_W = 2.0
TAU_W = 100.0
TAU_GE = 5.0
TAU_GI = 10.0
E_GE = 0.0
E_GI = -80.0
V_CUT = -30.0

NC = 2
NS = 16
NW = NC * NS
CHUNK = 2048
ROWS = CHUNK // 128
NCHUNKS = E // CHUNK
CPW = (NCHUNKS + NW - 1) // NW
CLS_STRIDE = 102400
CNT_LEN = 2 * CLS_STRIDE
ZSHARE = CNT_LEN // NS
ZCHUNK = 1600
INH_CODE = 2 * CLS_STRIDE + 1


def _code_body(spk_ref, exc_ref, code_ref):
    code_ref[...] = jnp.where(
        spk_ref[...],
        jnp.where(exc_ref[...], jnp.int32(1), jnp.int32(INH_CODE)),
        jnp.int32(0))


def _edge_body(code_hbm, ei_hbm, out_hbm,
               code_v, sd_v, idx_v0, idx_v1, ones_v,
               zbuf, cnt_sh, in_sem0, in_sem1, scat_sem):
    cid = lax.axis_index("c")
    sid = lax.axis_index("s")
    wid = sid * NC + cid
    in_sems = (in_sem0, in_sem1)
    idx_bufs = (idx_v0, idx_v1)

    def ofill(i, _):
        ones_v[pl.ds(i * 16, 16)] = jnp.ones((16,), jnp.float32)
        return 0
    lax.fori_loop(0, CHUNK // 16, ofill, 0)

    def zfill(i, _):
        zbuf[pl.ds(i * 16, 16)] = jnp.zeros((16,), jnp.float32)
        return 0
    lax.fori_loop(0, ZCHUNK // 16, zfill, 0)
    for k in range(ZSHARE // ZCHUNK):
        pltpu.sync_copy(zbuf, cnt_sh.at[pl.ds(sid * ZSHARE + k * ZCHUNK, ZCHUNK)])

    code_desc = pltpu.async_copy(code_hbm, code_v, scat_sem)
    for b in (0, 1):
        off = (wid + b * NW) * CHUNK
        pltpu.async_copy(ei_hbm.at[:, pl.ds(off, CHUNK)], sd_v.at[b], in_sems[b])
    code_desc.wait()
    plsc.subcore_barrier()

    def do_chunk(ci, b):
        g = wid + ci * NW
        idx_v = idx_bufs[b]

        @pl.when(g < NCHUNKS)
        def _process():
            off = g * CHUNK
            pltpu.make_async_copy(
                ei_hbm.at[:, pl.ds(off, CHUNK)], sd_v.at[b], in_sems[b]).wait()
            @pl.when(ci >= 2)
            def _drain():
                pltpu.make_async_copy(
                    ones_v,
                    cnt_sh.at[plsc.Indices(idx_v, ignored_value=-1)],
                    scat_sem).wait()

            @plsc.parallel_loop(0, CHUNK // 16, 1, unroll=8)
            def _vloop(i):
                s = sd_v[b, 0, pl.ds(i * 16, 16)]
                d = sd_v[b, 1, pl.ds(i * 16, 16)]
                t = plsc.load_gather(code_v, [s])
                slot = d + lax.shift_right_logical(t, 1)
                idx_v[pl.ds(i * 16, 16)] = jnp.where(
                    t == 0, jnp.int32(-1), slot)
            ng = g + 2 * NW

            @pl.when(ng < NCHUNKS)
            def _prefetch():
                noff = ng * CHUNK
                pltpu.async_copy(
                    ei_hbm.at[:, pl.ds(noff, CHUNK)], sd_v.at[b], in_sems[b])
            pltpu.async_copy(
                ones_v,
                cnt_sh.at[plsc.Indices(idx_v, ignored_value=-1)],
                scat_sem, add=True)

    def outer_body(o, _):
        do_chunk(2 * o, 0)
        do_chunk(2 * o + 1, 1)
        return 0
    lax.fori_loop(0, CPW // 2, outer_body, 0)

    for ci in (CPW - 2, CPW - 1):
        b = ci % 2

        @pl.when(wid + ci * NW < NCHUNKS)
        def _final_drain():
            pltpu.make_async_copy(
                ones_v,
                cnt_sh.at[plsc.Indices(idx_bufs[b], ignored_value=-1)],
                scat_sem).wait()

    plsc.subcore_barrier()
    pltpu.sync_copy(cnt_sh.at[pl.ds(sid * ZSHARE, ZSHARE)],
                    out_hbm.at[cid, pl.ds(sid * ZSHARE, ZSHARE)])


_edge_kernel = functools.partial(
    pl.kernel,
    out_type=jax.ShapeDtypeStruct((NC, CNT_LEN), jnp.float32),
    mesh=plsc.VectorSubcoreMesh(core_axis_name="c", subcore_axis_name="s"),
    compiler_params=pltpu.CompilerParams(needs_layout_passes=False),
    scratch_types=[
        pltpu.VMEM((N,), jnp.int32),
        pltpu.VMEM((2, 2, CHUNK), jnp.int32),
        pltpu.VMEM((CHUNK,), jnp.int32),
        pltpu.VMEM((CHUNK,), jnp.int32),
        pltpu.VMEM((CHUNK,), jnp.float32),
        pltpu.VMEM((ZCHUNK,), jnp.float32),
        pltpu.VMEM_SHARED((CNT_LEN,), jnp.float32),
        pltpu.SemaphoreType.DMA,
        pltpu.SemaphoreType.DMA,
        pltpu.SemaphoreType.DMA,
    ],
)(_edge_body)


def _adex_body(v_ref, st_ref, w_ref, ge_ref, gi_ref, qge_ref, qgi_ref,
               ib_ref, vr_ref, b_ref, tr_ref, rc_ref, pc_ref,
               out_ref, spk_ref):
    v = v_ref[...]
    w = w_ref[...]
    cnt_exc = pc_ref[0, :N] + pc_ref[1, :N]
    cnt_inh = (pc_ref[0, CLS_STRIDE:CLS_STRIDE + N]
               + pc_ref[1, CLS_STRIDE:CLS_STRIDE + N])
    ge = ge_ref[...] + qge_ref[...] * cnt_exc
    gi = gi_ref[...] + qgi_ref[...] * cnt_inh
    I = ib_ref[...] + st_ref[...] + ge * (E_GE - v) + gi * (E_GI - v)
    exp_term = G_L * DELTA_T * jnp.exp(
        jnp.minimum((v - V_THRESH) / DELTA_T, 20.0))
    dv = (-G_L * (v - V_REST) + exp_term - w + I) / CAP
    dw = (-w + A_W * (v - V_REST)) / TAU_W
    non_ref = rc_ref[...] <= 0.0
    v = jnp.where(non_ref, v + dv * DT, v)
    w = w + dw * DT
    ge = ge + (-ge / TAU_GE) * DT
    gi = gi + (-gi / TAU_GI) * DT
    new_spiked = v > V_CUT
    v = jnp.where(new_spiked, vr_ref[...], v)
    w = jnp.where(new_spiked, w + b_ref[...], w)
    refrac = jnp.where(new_spiked, tr_ref[...], rc_ref[...]) - DT
    out_ref[0, :] = v
    out_ref[1, :] = w
    out_ref[2, :] = ge
    out_ref[3, :] = gi
    out_ref[4, :] = refrac
    spk_ref[...] = new_spiked


def kernel(voltage, stimulus, adapt_current, ge, gi, Q_ge, Q_gi, I_bias,
           v_reset, b, t_refrac, refractory_counter, spiked, edge_index,
           is_excitatory):
    ei = edge_index.astype(jnp.int32)

    code = pl.pallas_call(
        _code_body,
        out_shape=jax.ShapeDtypeStruct((N,), jnp.int32),
    )(spiked, is_excitatory)

    pcounts = _edge_kernel(code, ei)

    out, new_spiked = pl.pallas_call(
        _adex_body,
        out_shape=(
            jax.ShapeDtypeStruct((5, N), jnp.float32),
            jax.ShapeDtypeStruct((N,), jnp.bool_),
        ),
    )(voltage, stimulus, adapt_current, ge, gi, Q_ge, Q_gi, I_bias,
      v_reset, b, t_refrac, refractory_counter, pcounts)
    return out, new_spiked
